# baseline (speedup 1.0000x reference)
import functools

import jax
import jax.numpy as jnp
from jax import lax
from jax.experimental import pallas as pl
from jax.experimental.pallas import tpu as pltpu
from jax.experimental.pallas import tpu_sc as plsc

_T1 = 1001
_D = 128
_B = 16384
_H = 20

_info = plsc.get_sparse_core_info()
_NC = _info.num_cores
_NS = _info.num_subcores
_NW = _NC * _NS
_RPW = _B // _NW

_CH = 128
_NCH = _RPW // _CH
_GPW = _H * _NCH
_NBUF = 5

_mesh = plsc.VectorSubcoreMesh(core_axis_name="c", subcore_axis_name="s")


@functools.partial(
    pl.kernel,
    mesh=_mesh,
    out_type=jax.ShapeDtypeStruct((_H, _B, _D), jnp.float32),
    scratch_types=[
        pltpu.VMEM((_H, _RPW), jnp.int32),
        pltpu.VMEM((_NBUF, _CH, _D), jnp.float32),
        pltpu.VMEM_SHARED((_T1, _D), jnp.float32),
        pltpu.SemaphoreType.DMA((_NBUF,)),
        pltpu.SemaphoreType.DMA((_NBUF,)),
    ],
)
def _sc_gather(idx_hbm, table_hbm, out_hbm, idx_v, rows_v, table_sp,
               gsem, ssem):
    sid = lax.axis_index("s")
    wid = sid * _NC + lax.axis_index("c")
    base = wid * _RPW

    @pl.when(sid < _NS - 1)
    def _():
        pltpu.sync_copy(table_hbm.at[pl.ds(sid * 64, 64)],
                        table_sp.at[pl.ds(sid * 64, 64)])

    @pl.when(sid == _NS - 1)
    def _():
        pltpu.sync_copy(table_hbm.at[pl.ds(960, _T1 - 960)],
                        table_sp.at[pl.ds(960, _T1 - 960)])

    pltpu.sync_copy(idx_hbm.at[pl.ds(0, _H), pl.ds(base, _RPW)], idx_v)
    plsc.subcore_barrier()

    def fire_gather(slot, g):
        h = g // _NCH
        c = g % _NCH
        return pltpu.async_copy(
            table_sp.at[idx_v.at[h, pl.ds(c * _CH, _CH)]], rows_v.at[slot],
            gsem.at[slot])

    def fire_store(slot, g):
        h = g // _NCH
        c = g % _NCH
        pltpu.async_copy(
            rows_v.at[slot], out_hbm.at[h, pl.ds(base + c * _CH, _CH)],
            ssem.at[slot])

    def wait_store(slot, g):
        h = g // _NCH
        c = g % _NCH
        pltpu.make_async_copy(
            rows_v.at[slot], out_hbm.at[h, pl.ds(base + c * _CH, _CH)],
            ssem.at[slot]).wait()

    def outer(o, carry):
        handles = []
        for b in range(_NBUF):
            g = o * _NBUF + b

            @pl.when(g >= _NBUF)
            def _():
                wait_store(b, g - _NBUF)

            handles.append(fire_gather(b, g))
        for b in range(_NBUF):
            g = o * _NBUF + b
            handles[b].wait()
            fire_store(b, g)
        return carry

    lax.fori_loop(0, _GPW // _NBUF, outer, 0)

    for g in range(_GPW - _NBUF, _GPW):
        wait_store(g % _NBUF, g)


def kernel(inputs, time_encodings):
    out_hbd = _sc_gather(inputs.T, time_encodings)
    return jnp.transpose(out_hbd, (1, 0, 2))
